# baseline (device time: 29007 ns/iter reference)
import jax
import jax.numpy as jnp
from jax import lax
from jax.experimental import pallas as pl
from jax.experimental.pallas import tpu as pltpu

N_DEV = 4


def kernel(A, B):
    m, k = A.shape
    _, n = B.shape

    def body(a_ref, b_ref, out_ref, comm_ref, send_sems, recv_sems):
        my = lax.axis_index("i")
        left = lax.rem(my + (N_DEV - 1), N_DEV)
        right = lax.rem(my + 1, N_DEV)

        barrier_sem = pltpu.get_barrier_semaphore()
        for nbr in [left, right]:
            pl.semaphore_signal(
                barrier_sem, inc=1,
                device_id=(nbr,), device_id_type=pl.DeviceIdType.MESH,
            )
        pl.semaphore_wait(barrier_sem, 2)

        partial = jnp.dot(
            a_ref[:, :].astype(jnp.bfloat16),
            b_ref[:, :].astype(jnp.bfloat16),
            preferred_element_type=jnp.float32,
        )
        comm_ref[0, :, :] = partial.astype(jnp.bfloat16)
        acc = partial

        for h in range(N_DEV - 1):
            rdma = pltpu.make_async_remote_copy(
                src_ref=comm_ref.at[h],
                dst_ref=comm_ref.at[h + 1],
                send_sem=send_sems.at[h],
                recv_sem=recv_sems.at[h + 1],
                device_id=(right,),
                device_id_type=pl.DeviceIdType.MESH,
            )
            rdma.start()
            rdma.wait()
            acc = acc + comm_ref[h + 1, :, :].astype(jnp.float32)

        z = acc
        out_ref[:, :] = 0.5 * z * (
            1.0 + jnp.tanh(0.7978845608 * (z + 0.044715 * z * z * z))
        )

    return pl.pallas_call(
        body,
        out_shape=jax.ShapeDtypeStruct((m, n), jnp.float32),
        in_specs=[
            pl.BlockSpec(memory_space=pltpu.VMEM),
            pl.BlockSpec(memory_space=pltpu.VMEM),
        ],
        out_specs=pl.BlockSpec(memory_space=pltpu.VMEM),
        scratch_shapes=[
            pltpu.VMEM((N_DEV, m, n), jnp.bfloat16),
            pltpu.SemaphoreType.DMA((N_DEV,)),
            pltpu.SemaphoreType.DMA((N_DEV,)),
        ],
        compiler_params=pltpu.CompilerParams(collective_id=0),
    )(A, B)


# device time: 16392 ns/iter; 1.7696x vs baseline; 1.7696x over previous
import jax
import jax.numpy as jnp
from jax import lax
from jax.experimental import pallas as pl
from jax.experimental.pallas import tpu as pltpu

N_DEV = 4


def kernel(A, B):
    m, k = A.shape
    _, n = B.shape
    mq = m // N_DEV

    def body(a_ref, b_ref, out_ref, blocks_ref, rs_buf, ag_src, ag_buf,
             rs_send_sems, rs_recv_sems, ag_send_sems, ag_recv_sems):
        my = lax.axis_index("i")

        barrier_sem = pltpu.get_barrier_semaphore()
        for d in (1, N_DEV - 1):
            pl.semaphore_signal(
                barrier_sem, inc=1,
                device_id=(lax.rem(my + d, N_DEV),),
                device_id_type=pl.DeviceIdType.MESH,
            )
        pl.semaphore_wait(barrier_sem, 2)

        partial = jnp.dot(
            a_ref[:, :].astype(jnp.bfloat16),
            b_ref[:, :].astype(jnp.bfloat16),
            preferred_element_type=jnp.float32,
        )
        blocks_ref[:, :, :] = partial.astype(jnp.bfloat16).reshape(
            N_DEV, mq, n
        )

        rs_sends = []
        for d in range(1, N_DEV):
            tgt = lax.rem(my + d, N_DEV)
            rdma = pltpu.make_async_remote_copy(
                src_ref=blocks_ref.at[tgt],
                dst_ref=rs_buf.at[N_DEV - d],
                send_sem=rs_send_sems.at[d],
                recv_sem=rs_recv_sems.at[N_DEV - d],
                device_id=(tgt,),
                device_id_type=pl.DeviceIdType.MESH,
            )
            rdma.start()
            rs_sends.append(rdma)

        acc = blocks_ref[my, :, :].astype(jnp.float32)

        for kk in range(1, N_DEV):
            recv = pltpu.make_async_remote_copy(
                src_ref=blocks_ref.at[0],
                dst_ref=rs_buf.at[kk],
                send_sem=rs_send_sems.at[0],
                recv_sem=rs_recv_sems.at[kk],
                device_id=(my,),
                device_id_type=pl.DeviceIdType.MESH,
            )
            recv.wait_recv()
            acc = acc + rs_buf[kk, :, :].astype(jnp.float32)

        z = acc
        g = 0.5 * z * (1.0 + jnp.tanh(0.7978845608 * (z + 0.044715 * z * z * z)))
        ag_src[:, :] = g.astype(jnp.bfloat16)
        out_ref[pl.ds(my * mq, mq), :] = g

        ag_sends = []
        for d in range(1, N_DEV):
            tgt = lax.rem(my + d, N_DEV)
            rdma = pltpu.make_async_remote_copy(
                src_ref=ag_src,
                dst_ref=ag_buf.at[N_DEV - d],
                send_sem=ag_send_sems.at[d],
                recv_sem=ag_recv_sems.at[N_DEV - d],
                device_id=(tgt,),
                device_id_type=pl.DeviceIdType.MESH,
            )
            rdma.start()
            ag_sends.append(rdma)

        for kk in range(1, N_DEV):
            recv = pltpu.make_async_remote_copy(
                src_ref=ag_src,
                dst_ref=ag_buf.at[kk],
                send_sem=ag_send_sems.at[0],
                recv_sem=ag_recv_sems.at[kk],
                device_id=(my,),
                device_id_type=pl.DeviceIdType.MESH,
            )
            recv.wait_recv()
            origin = lax.rem(my + kk, N_DEV)
            out_ref[pl.ds(origin * mq, mq), :] = (
                ag_buf[kk, :, :].astype(jnp.float32)
            )

        for rdma in rs_sends + ag_sends:
            rdma.wait_send()

    return pl.pallas_call(
        body,
        out_shape=jax.ShapeDtypeStruct((m, n), jnp.float32),
        in_specs=[
            pl.BlockSpec(memory_space=pltpu.VMEM),
            pl.BlockSpec(memory_space=pltpu.VMEM),
        ],
        out_specs=pl.BlockSpec(memory_space=pltpu.VMEM),
        scratch_shapes=[
            pltpu.VMEM((N_DEV, mq, n), jnp.bfloat16),
            pltpu.VMEM((N_DEV, mq, n), jnp.bfloat16),
            pltpu.VMEM((mq, n), jnp.bfloat16),
            pltpu.VMEM((N_DEV, mq, n), jnp.bfloat16),
            pltpu.SemaphoreType.DMA((N_DEV,)),
            pltpu.SemaphoreType.DMA((N_DEV,)),
            pltpu.SemaphoreType.DMA((N_DEV,)),
            pltpu.SemaphoreType.DMA((N_DEV,)),
        ],
        compiler_params=pltpu.CompilerParams(collective_id=0),
    )(A, B)


# device time: 16033 ns/iter; 1.8092x vs baseline; 1.0224x over previous
import jax
import jax.numpy as jnp
from jax import lax
from jax.experimental import pallas as pl
from jax.experimental.pallas import tpu as pltpu

N_DEV = 4
SEND_ORDER = (2, 1, 3)
RECV_ORDER = (1, 3, 2)


def kernel(A, B):
    m, k = A.shape
    _, n = B.shape
    mq = m // N_DEV

    def body(a_ref, b_ref, out_ref, blocks_ref, rs_buf, ag_src, ag_buf,
             rs_send_sems, rs_recv_sems, ag_send_sems, ag_recv_sems):
        my = lax.axis_index("i")

        barrier_sem = pltpu.get_barrier_semaphore()
        for d in range(1, N_DEV):
            pl.semaphore_signal(
                barrier_sem, inc=1,
                device_id=(lax.rem(my + d, N_DEV),),
                device_id_type=pl.DeviceIdType.MESH,
            )

        partial = jnp.dot(
            a_ref[:, :].astype(jnp.bfloat16),
            b_ref[:, :].astype(jnp.bfloat16),
            preferred_element_type=jnp.float32,
        )
        blocks_ref[:, :, :] = partial.astype(jnp.bfloat16).reshape(
            N_DEV, mq, n
        )

        pl.semaphore_wait(barrier_sem, N_DEV - 1)

        rs_sends = []
        for d in SEND_ORDER:
            tgt = lax.rem(my + d, N_DEV)
            rdma = pltpu.make_async_remote_copy(
                src_ref=blocks_ref.at[tgt],
                dst_ref=rs_buf.at[N_DEV - d],
                send_sem=rs_send_sems.at[d],
                recv_sem=rs_recv_sems.at[N_DEV - d],
                device_id=(tgt,),
                device_id_type=pl.DeviceIdType.MESH,
            )
            rdma.start()
            rs_sends.append(rdma)

        acc = blocks_ref[my, :, :].astype(jnp.float32)

        for kk in RECV_ORDER:
            recv = pltpu.make_async_remote_copy(
                src_ref=blocks_ref.at[0],
                dst_ref=rs_buf.at[kk],
                send_sem=rs_send_sems.at[0],
                recv_sem=rs_recv_sems.at[kk],
                device_id=(my,),
                device_id_type=pl.DeviceIdType.MESH,
            )
            recv.wait_recv()
            acc = acc + rs_buf[kk, :, :].astype(jnp.float32)

        z = acc
        g = 0.5 * z * (1.0 + jnp.tanh(0.7978845608 * (z + 0.044715 * z * z * z)))
        ag_src[:, :] = g.astype(jnp.bfloat16)

        ag_sends = []
        for d in SEND_ORDER:
            tgt = lax.rem(my + d, N_DEV)
            rdma = pltpu.make_async_remote_copy(
                src_ref=ag_src,
                dst_ref=ag_buf.at[N_DEV - d],
                send_sem=ag_send_sems.at[d],
                recv_sem=ag_recv_sems.at[N_DEV - d],
                device_id=(tgt,),
                device_id_type=pl.DeviceIdType.MESH,
            )
            rdma.start()
            ag_sends.append(rdma)

        out_ref[pl.ds(my * mq, mq), :] = g

        for kk in RECV_ORDER:
            recv = pltpu.make_async_remote_copy(
                src_ref=ag_src,
                dst_ref=ag_buf.at[kk],
                send_sem=ag_send_sems.at[0],
                recv_sem=ag_recv_sems.at[kk],
                device_id=(my,),
                device_id_type=pl.DeviceIdType.MESH,
            )
            recv.wait_recv()
            origin = lax.rem(my + kk, N_DEV)
            out_ref[pl.ds(origin * mq, mq), :] = (
                ag_buf[kk, :, :].astype(jnp.float32)
            )

        for rdma in rs_sends + ag_sends:
            rdma.wait_send()

    return pl.pallas_call(
        body,
        out_shape=jax.ShapeDtypeStruct((m, n), jnp.float32),
        in_specs=[
            pl.BlockSpec(memory_space=pltpu.VMEM),
            pl.BlockSpec(memory_space=pltpu.VMEM),
        ],
        out_specs=pl.BlockSpec(memory_space=pltpu.VMEM),
        scratch_shapes=[
            pltpu.VMEM((N_DEV, mq, n), jnp.bfloat16),
            pltpu.VMEM((N_DEV, mq, n), jnp.bfloat16),
            pltpu.VMEM((mq, n), jnp.bfloat16),
            pltpu.VMEM((N_DEV, mq, n), jnp.bfloat16),
            pltpu.SemaphoreType.DMA((N_DEV,)),
            pltpu.SemaphoreType.DMA((N_DEV,)),
            pltpu.SemaphoreType.DMA((N_DEV,)),
            pltpu.SemaphoreType.DMA((N_DEV,)),
        ],
        compiler_params=pltpu.CompilerParams(collective_id=0),
    )(A, B)


# device time: 15837 ns/iter; 1.8316x vs baseline; 1.0124x over previous
import jax
import jax.numpy as jnp
from jax import lax
from jax.experimental import pallas as pl
from jax.experimental.pallas import tpu as pltpu

N_DEV = 4
SEND_ORDER = (2, 1, 3)
RECV_ORDER = (1, 3, 2)


def kernel(A, B):
    m, k = A.shape
    _, n = B.shape
    mq = m // N_DEV

    def body(a_ref, b_ref, out_ref, blocks_ref, rs_buf,
             rs_send_sems, rs_recv_sems, ag_send_sems, ag_recv_sems):
        my = lax.axis_index("i")

        barrier_sem = pltpu.get_barrier_semaphore()
        for d in range(1, N_DEV):
            pl.semaphore_signal(
                barrier_sem, inc=1,
                device_id=(lax.rem(my + d, N_DEV),),
                device_id_type=pl.DeviceIdType.MESH,
            )

        partial = jnp.dot(
            a_ref[:, :].astype(jnp.bfloat16),
            b_ref[:, :].astype(jnp.bfloat16),
            preferred_element_type=jnp.float32,
        )
        blocks_ref[:, :, :] = partial.astype(jnp.bfloat16).reshape(
            N_DEV, mq, n
        )

        pl.semaphore_wait(barrier_sem, N_DEV - 1)

        rs_sends = []
        for d in SEND_ORDER:
            tgt = lax.rem(my + d, N_DEV)
            rdma = pltpu.make_async_remote_copy(
                src_ref=blocks_ref.at[tgt],
                dst_ref=rs_buf.at[N_DEV - d],
                send_sem=rs_send_sems.at[d],
                recv_sem=rs_recv_sems.at[N_DEV - d],
                device_id=(tgt,),
                device_id_type=pl.DeviceIdType.MESH,
            )
            rdma.start()
            rs_sends.append(rdma)

        acc = blocks_ref[my, :, :].astype(jnp.float32)

        for kk in RECV_ORDER:
            recv = pltpu.make_async_remote_copy(
                src_ref=blocks_ref.at[0],
                dst_ref=rs_buf.at[kk],
                send_sem=rs_send_sems.at[0],
                recv_sem=rs_recv_sems.at[kk],
                device_id=(my,),
                device_id_type=pl.DeviceIdType.MESH,
            )
            recv.wait_recv()
            acc = acc + rs_buf[kk, :, :].astype(jnp.float32)

        z = acc
        g = 0.5 * z * (1.0 + jnp.tanh(0.7978845608 * (z + 0.044715 * z * z * z)))
        my_rows = out_ref.at[pl.ds(my * mq, mq), :]
        my_rows[:, :] = g.astype(jnp.bfloat16)

        ag_sends = []
        for d in SEND_ORDER:
            tgt = lax.rem(my + d, N_DEV)
            rdma = pltpu.make_async_remote_copy(
                src_ref=my_rows,
                dst_ref=my_rows,
                send_sem=ag_send_sems.at[d],
                recv_sem=ag_recv_sems.at[N_DEV - d],
                device_id=(tgt,),
                device_id_type=pl.DeviceIdType.MESH,
            )
            rdma.start()
            ag_sends.append(rdma)

        for kk in RECV_ORDER:
            recv = pltpu.make_async_remote_copy(
                src_ref=my_rows,
                dst_ref=my_rows,
                send_sem=ag_send_sems.at[0],
                recv_sem=ag_recv_sems.at[kk],
                device_id=(my,),
                device_id_type=pl.DeviceIdType.MESH,
            )
            recv.wait_recv()

        for rdma in rs_sends + ag_sends:
            rdma.wait_send()

    return pl.pallas_call(
        body,
        out_shape=jax.ShapeDtypeStruct((m, n), jnp.bfloat16),
        in_specs=[
            pl.BlockSpec(memory_space=pltpu.VMEM),
            pl.BlockSpec(memory_space=pltpu.VMEM),
        ],
        out_specs=pl.BlockSpec(memory_space=pltpu.VMEM),
        scratch_shapes=[
            pltpu.VMEM((N_DEV, mq, n), jnp.bfloat16),
            pltpu.VMEM((N_DEV, mq, n), jnp.bfloat16),
            pltpu.SemaphoreType.DMA((N_DEV,)),
            pltpu.SemaphoreType.DMA((N_DEV,)),
            pltpu.SemaphoreType.DMA((N_DEV,)),
            pltpu.SemaphoreType.DMA((N_DEV,)),
        ],
        compiler_params=pltpu.CompilerParams(collective_id=0),
    )(A, B)


# device time: 14469 ns/iter; 2.0048x vs baseline; 1.0945x over previous
import jax
import jax.numpy as jnp
from jax import lax
from jax.experimental import pallas as pl
from jax.experimental.pallas import tpu as pltpu

N_DEV = 4
C = 2
SEND_ORDER = (2, 1, 3)
RECV_ORDER = (1, 3, 2)


def kernel(A, B):
    m, k = A.shape
    _, n = B.shape
    mq = m // N_DEV
    r = mq // C

    def body(a_ref, b_ref, out_ref, blocks_ref, rs_buf,
             rs_send_sems, rs_recv_sems, ag_send_sems, ag_recv_sems):
        my = lax.axis_index("i")

        barrier_sem = pltpu.get_barrier_semaphore()
        for d in range(1, N_DEV):
            pl.semaphore_signal(
                barrier_sem, inc=1,
                device_id=(lax.rem(my + d, N_DEV),),
                device_id_type=pl.DeviceIdType.MESH,
            )

        partial = jnp.dot(
            a_ref[:, :].astype(jnp.bfloat16),
            b_ref[:, :].astype(jnp.bfloat16),
            preferred_element_type=jnp.float32,
        )
        blocks_ref[:, :, :] = partial.astype(jnp.bfloat16).reshape(
            N_DEV, mq, n
        )

        pl.semaphore_wait(barrier_sem, N_DEV - 1)

        rs_sends = []
        for c in range(C):
            for d in SEND_ORDER:
                tgt = lax.rem(my + d, N_DEV)
                rdma = pltpu.make_async_remote_copy(
                    src_ref=blocks_ref.at[tgt, pl.ds(c * r, r), :],
                    dst_ref=rs_buf.at[N_DEV - d, pl.ds(c * r, r), :],
                    send_sem=rs_send_sems.at[d, c],
                    recv_sem=rs_recv_sems.at[N_DEV - d, c],
                    device_id=(tgt,),
                    device_id_type=pl.DeviceIdType.MESH,
                )
                rdma.start()
                rs_sends.append(rdma)

        ag_sends = []
        for c in range(C):
            acc = blocks_ref[my, pl.ds(c * r, r), :].astype(jnp.float32)
            for kk in RECV_ORDER:
                recv = pltpu.make_async_remote_copy(
                    src_ref=blocks_ref.at[0, pl.ds(c * r, r), :],
                    dst_ref=rs_buf.at[kk, pl.ds(c * r, r), :],
                    send_sem=rs_send_sems.at[0, c],
                    recv_sem=rs_recv_sems.at[kk, c],
                    device_id=(my,),
                    device_id_type=pl.DeviceIdType.MESH,
                )
                recv.wait_recv()
                acc = acc + rs_buf[kk, pl.ds(c * r, r), :].astype(jnp.float32)

            z = acc
            g = 0.5 * z * (
                1.0 + jnp.tanh(0.7978845608 * (z + 0.044715 * z * z * z))
            )
            my_rows = out_ref.at[pl.ds(my * mq + c * r, r), :]
            my_rows[:, :] = g.astype(jnp.bfloat16)

            for d in SEND_ORDER:
                tgt = lax.rem(my + d, N_DEV)
                rdma = pltpu.make_async_remote_copy(
                    src_ref=my_rows,
                    dst_ref=my_rows,
                    send_sem=ag_send_sems.at[d, c],
                    recv_sem=ag_recv_sems.at[N_DEV - d, c],
                    device_id=(tgt,),
                    device_id_type=pl.DeviceIdType.MESH,
                )
                rdma.start()
                ag_sends.append(rdma)

        for c in range(C):
            for kk in RECV_ORDER:
                recv = pltpu.make_async_remote_copy(
                    src_ref=out_ref.at[pl.ds(my * mq + c * r, r), :],
                    dst_ref=out_ref.at[pl.ds(my * mq + c * r, r), :],
                    send_sem=ag_send_sems.at[0, c],
                    recv_sem=ag_recv_sems.at[kk, c],
                    device_id=(my,),
                    device_id_type=pl.DeviceIdType.MESH,
                )
                recv.wait_recv()

        for rdma in rs_sends + ag_sends:
            rdma.wait_send()

    return pl.pallas_call(
        body,
        out_shape=jax.ShapeDtypeStruct((m, n), jnp.bfloat16),
        in_specs=[
            pl.BlockSpec(memory_space=pltpu.VMEM),
            pl.BlockSpec(memory_space=pltpu.VMEM),
        ],
        out_specs=pl.BlockSpec(memory_space=pltpu.VMEM),
        scratch_shapes=[
            pltpu.VMEM((N_DEV, mq, n), jnp.bfloat16),
            pltpu.VMEM((N_DEV, mq, n), jnp.bfloat16),
            pltpu.SemaphoreType.DMA((N_DEV, C)),
            pltpu.SemaphoreType.DMA((N_DEV, C)),
            pltpu.SemaphoreType.DMA((N_DEV, C)),
            pltpu.SemaphoreType.DMA((N_DEV, C)),
        ],
        compiler_params=pltpu.CompilerParams(collective_id=0),
    )(A, B)


# device time: 14299 ns/iter; 2.0286x vs baseline; 1.0119x over previous
import jax
import jax.numpy as jnp
from jax import lax
from jax.experimental import pallas as pl
from jax.experimental.pallas import tpu as pltpu

N_DEV = 4
C = 4
SEND_ORDER = (2, 1, 3)
RECV_ORDER = (1, 3, 2)


def kernel(A, B):
    m, k = A.shape
    _, n = B.shape
    mq = m // N_DEV
    r = mq // C

    def body(a_ref, b_ref, out_ref, blocks_ref, rs_buf,
             rs_send_sems, rs_recv_sems, ag_send_sems, ag_recv_sems):
        my = lax.axis_index("i")

        barrier_sem = pltpu.get_barrier_semaphore()
        for d in range(1, N_DEV):
            pl.semaphore_signal(
                barrier_sem, inc=1,
                device_id=(lax.rem(my + d, N_DEV),),
                device_id_type=pl.DeviceIdType.MESH,
            )

        partial = jnp.dot(
            a_ref[:, :].astype(jnp.bfloat16),
            b_ref[:, :].astype(jnp.bfloat16),
            preferred_element_type=jnp.float32,
        )
        blocks_ref[:, :, :] = partial.astype(jnp.bfloat16).reshape(
            N_DEV, mq, n
        )

        pl.semaphore_wait(barrier_sem, N_DEV - 1)

        rs_sends = []
        for c in range(C):
            for d in SEND_ORDER:
                tgt = lax.rem(my + d, N_DEV)
                rdma = pltpu.make_async_remote_copy(
                    src_ref=blocks_ref.at[tgt, pl.ds(c * r, r), :],
                    dst_ref=rs_buf.at[N_DEV - d, pl.ds(c * r, r), :],
                    send_sem=rs_send_sems.at[d, c],
                    recv_sem=rs_recv_sems.at[N_DEV - d, c],
                    device_id=(tgt,),
                    device_id_type=pl.DeviceIdType.MESH,
                )
                rdma.start()
                rs_sends.append(rdma)

        ag_sends = []
        for c in range(C):
            acc = blocks_ref[my, pl.ds(c * r, r), :].astype(jnp.float32)
            for kk in RECV_ORDER:
                recv = pltpu.make_async_remote_copy(
                    src_ref=blocks_ref.at[0, pl.ds(c * r, r), :],
                    dst_ref=rs_buf.at[kk, pl.ds(c * r, r), :],
                    send_sem=rs_send_sems.at[0, c],
                    recv_sem=rs_recv_sems.at[kk, c],
                    device_id=(my,),
                    device_id_type=pl.DeviceIdType.MESH,
                )
                recv.wait_recv()
                acc = acc + rs_buf[kk, pl.ds(c * r, r), :].astype(jnp.float32)

            z = acc
            g = 0.5 * z * (
                1.0 + jnp.tanh(0.7978845608 * (z + 0.044715 * z * z * z))
            )
            my_rows = out_ref.at[pl.ds(my * mq + c * r, r), :]
            my_rows[:, :] = g.astype(jnp.bfloat16)

            for d in SEND_ORDER:
                tgt = lax.rem(my + d, N_DEV)
                rdma = pltpu.make_async_remote_copy(
                    src_ref=my_rows,
                    dst_ref=my_rows,
                    send_sem=ag_send_sems.at[d, c],
                    recv_sem=ag_recv_sems.at[N_DEV - d, c],
                    device_id=(tgt,),
                    device_id_type=pl.DeviceIdType.MESH,
                )
                rdma.start()
                ag_sends.append(rdma)

        for c in range(C):
            for kk in RECV_ORDER:
                recv = pltpu.make_async_remote_copy(
                    src_ref=out_ref.at[pl.ds(my * mq + c * r, r), :],
                    dst_ref=out_ref.at[pl.ds(my * mq + c * r, r), :],
                    send_sem=ag_send_sems.at[0, c],
                    recv_sem=ag_recv_sems.at[kk, c],
                    device_id=(my,),
                    device_id_type=pl.DeviceIdType.MESH,
                )
                recv.wait_recv()

        for rdma in rs_sends + ag_sends:
            rdma.wait_send()

    return pl.pallas_call(
        body,
        out_shape=jax.ShapeDtypeStruct((m, n), jnp.bfloat16),
        in_specs=[
            pl.BlockSpec(memory_space=pltpu.VMEM),
            pl.BlockSpec(memory_space=pltpu.VMEM),
        ],
        out_specs=pl.BlockSpec(memory_space=pltpu.VMEM),
        scratch_shapes=[
            pltpu.VMEM((N_DEV, mq, n), jnp.bfloat16),
            pltpu.VMEM((N_DEV, mq, n), jnp.bfloat16),
            pltpu.SemaphoreType.DMA((N_DEV, C)),
            pltpu.SemaphoreType.DMA((N_DEV, C)),
            pltpu.SemaphoreType.DMA((N_DEV, C)),
            pltpu.SemaphoreType.DMA((N_DEV, C)),
        ],
        compiler_params=pltpu.CompilerParams(collective_id=0),
    )(A, B)


# device time: 13937 ns/iter; 2.0813x vs baseline; 1.0260x over previous
import jax
import jax.numpy as jnp
from jax import lax
from jax.experimental import pallas as pl
from jax.experimental.pallas import tpu as pltpu

N_DEV = 4
C = 2
SEND_ORDER = (2, 1, 3)
RECV_ORDER = (1, 3, 2)


def kernel(A, B):
    m, k = A.shape
    _, n = B.shape
    mq = m // N_DEV
    r = mq // C

    def body(a_hbm, b_hbm, out_ref, a_v, b_v, blocks_ref, rs_buf, ag_src,
             rs_send_sems, rs_recv_sems, ag_send_sems, ag_recv_sems,
             local_sems):
        my = lax.axis_index("i")

        cp_a = pltpu.make_async_copy(a_hbm, a_v, local_sems.at[0])
        cp_b = pltpu.make_async_copy(b_hbm, b_v, local_sems.at[1])
        cp_a.start()
        cp_b.start()

        barrier_sem = pltpu.get_barrier_semaphore()
        for d in range(1, N_DEV):
            pl.semaphore_signal(
                barrier_sem, inc=1,
                device_id=(lax.rem(my + d, N_DEV),),
                device_id_type=pl.DeviceIdType.MESH,
            )

        cp_a.wait()
        cp_b.wait()

        partial = jnp.dot(
            a_v[:, :].astype(jnp.bfloat16),
            b_v[:, :].astype(jnp.bfloat16),
            preferred_element_type=jnp.float32,
        )
        blocks_ref[:, :, :] = partial.astype(jnp.bfloat16).reshape(
            N_DEV, mq, n
        )

        pl.semaphore_wait(barrier_sem, N_DEV - 1)

        rs_sends = []
        for c in range(C):
            for d in SEND_ORDER:
                tgt = lax.rem(my + d, N_DEV)
                rdma = pltpu.make_async_remote_copy(
                    src_ref=blocks_ref.at[tgt, pl.ds(c * r, r), :],
                    dst_ref=rs_buf.at[N_DEV - d, pl.ds(c * r, r), :],
                    send_sem=rs_send_sems.at[d, c],
                    recv_sem=rs_recv_sems.at[N_DEV - d, c],
                    device_id=(tgt,),
                    device_id_type=pl.DeviceIdType.MESH,
                )
                rdma.start()
                rs_sends.append(rdma)

        ag_sends = []
        out_copies = []
        for c in range(C):
            acc = blocks_ref[my, pl.ds(c * r, r), :].astype(jnp.float32)
            for kk in RECV_ORDER:
                recv = pltpu.make_async_remote_copy(
                    src_ref=blocks_ref.at[0, pl.ds(c * r, r), :],
                    dst_ref=rs_buf.at[kk, pl.ds(c * r, r), :],
                    send_sem=rs_send_sems.at[0, c],
                    recv_sem=rs_recv_sems.at[kk, c],
                    device_id=(my,),
                    device_id_type=pl.DeviceIdType.MESH,
                )
                recv.wait_recv()
                acc = acc + rs_buf[kk, pl.ds(c * r, r), :].astype(jnp.float32)

            z = acc
            g = 0.5 * z * (
                1.0 + jnp.tanh(0.7978845608 * (z + 0.044715 * z * z * z))
            )
            src_rows = ag_src.at[pl.ds(c * r, r), :]
            src_rows[:, :] = g.astype(jnp.bfloat16)
            dst_rows = out_ref.at[pl.ds(my * mq + c * r, r), :]

            for d in SEND_ORDER:
                tgt = lax.rem(my + d, N_DEV)
                rdma = pltpu.make_async_remote_copy(
                    src_ref=src_rows,
                    dst_ref=dst_rows,
                    send_sem=ag_send_sems.at[d, c],
                    recv_sem=ag_recv_sems.at[N_DEV - d, c],
                    device_id=(tgt,),
                    device_id_type=pl.DeviceIdType.MESH,
                )
                rdma.start()
                ag_sends.append(rdma)

            cp = pltpu.make_async_copy(src_rows, dst_rows,
                                       local_sems.at[2 + c])
            cp.start()
            out_copies.append(cp)

        for c in range(C):
            for kk in RECV_ORDER:
                recv = pltpu.make_async_remote_copy(
                    src_ref=ag_src.at[pl.ds(c * r, r), :],
                    dst_ref=out_ref.at[pl.ds(my * mq + c * r, r), :],
                    send_sem=ag_send_sems.at[0, c],
                    recv_sem=ag_recv_sems.at[kk, c],
                    device_id=(my,),
                    device_id_type=pl.DeviceIdType.MESH,
                )
                recv.wait_recv()

        for cp in out_copies:
            cp.wait()
        for rdma in rs_sends + ag_sends:
            rdma.wait_send()

    return pl.pallas_call(
        body,
        out_shape=jax.ShapeDtypeStruct((m, n), jnp.bfloat16),
        in_specs=[
            pl.BlockSpec(memory_space=pl.ANY),
            pl.BlockSpec(memory_space=pl.ANY),
        ],
        out_specs=pl.BlockSpec(memory_space=pl.ANY),
        scratch_shapes=[
            pltpu.VMEM((m, k), jnp.float32),
            pltpu.VMEM((k, n), jnp.float32),
            pltpu.VMEM((N_DEV, mq, n), jnp.bfloat16),
            pltpu.VMEM((N_DEV, mq, n), jnp.bfloat16),
            pltpu.VMEM((mq, n), jnp.bfloat16),
            pltpu.SemaphoreType.DMA((N_DEV, C)),
            pltpu.SemaphoreType.DMA((N_DEV, C)),
            pltpu.SemaphoreType.DMA((N_DEV, C)),
            pltpu.SemaphoreType.DMA((N_DEV, C)),
            pltpu.SemaphoreType.DMA((2 + C,)),
        ],
        compiler_params=pltpu.CompilerParams(collective_id=0),
    )(A, B)
